# baseline (speedup 1.0000x reference)
import functools

import jax
import jax.numpy as jnp
from jax import lax
from jax.experimental import pallas as pl
from jax.experimental.pallas import tpu as pltpu
from jax.experimental.pallas import tpu_sc as plsc

PAIR_DIM = 128
EMBED_DIM = 64
LANES = 16
NUM_CORES = 2
NUM_SUBCORES = 16
NUM_WORKERS = NUM_CORES * NUM_SUBCORES
CHUNK = 160
N_CHUNKS = 40

TBLOCK = 2048


def _transpose_block(in_ref, out_ref):
    x = in_ref[...]
    y = x.T.reshape(TBLOCK, 2, EMBED_DIM)
    out_ref[...] = jnp.concatenate([y[:, 0, :], y[:, 1, :]], axis=1)


def _relayout_table(wt):
    v = wt.shape[1]
    grid = (v // 2 + TBLOCK - 1) // TBLOCK
    return pl.pallas_call(
        _transpose_block,
        grid=(grid,),
        in_specs=[pl.BlockSpec((EMBED_DIM, 2 * TBLOCK), lambda n: (0, n))],
        out_specs=pl.BlockSpec((TBLOCK, PAIR_DIM), lambda n: (n, 0)),
        out_shape=jax.ShapeDtypeStruct((v // 2, PAIR_DIM), jnp.float32),
    )(wt)


def _make_gather(total_rows: int):
    rows_per_w = total_rows // NUM_WORKERS
    assert rows_per_w == CHUNK * N_CHUNKS
    mesh = plsc.VectorSubcoreMesh(core_axis_name="c", subcore_axis_name="s")

    @functools.partial(
        pl.kernel,
        mesh=mesh,
        out_type=jax.ShapeDtypeStruct((total_rows, EMBED_DIM), jnp.float32),
        scratch_types=[
            pltpu.VMEM((rows_per_w,), jnp.int32),
            pltpu.VMEM((rows_per_w,), jnp.int32),
            pltpu.VMEM((CHUNK, PAIR_DIM), jnp.float32),
            pltpu.VMEM((CHUNK, PAIR_DIM), jnp.float32),
            pltpu.VMEM((CHUNK, EMBED_DIM), jnp.float32),
            pltpu.VMEM((CHUNK, EMBED_DIM), jnp.float32),
            pltpu.SemaphoreType.DMA,
            pltpu.SemaphoreType.DMA,
            pltpu.SemaphoreType.DMA,
            pltpu.SemaphoreType.DMA,
        ],
    )
    def gather(
        table_hbm, idx_hbm, out_hbm,
        idx_v, pair_v, rows0, rows1, comp0, comp1, g0, g1, o0, o1,
    ):
        wid = lax.axis_index("s") * NUM_CORES + lax.axis_index("c")
        base = wid * rows_per_w
        pltpu.sync_copy(idx_hbm.at[pl.ds(base, rows_per_w)], idx_v)

        def halve(j, carry):
            idx16 = idx_v[pl.ds(j * LANES, LANES)]
            pair_v[pl.ds(j * LANES, LANES)] = lax.shift_right_logical(idx16, 1)
            return carry

        lax.fori_loop(0, rows_per_w // LANES, halve, 0)

        rows = (rows0, rows1)
        comp = (comp0, comp1)
        gsem = (g0, g1)
        osem = (o0, o1)

        def start_gather(i):
            return pltpu.async_copy(
                table_hbm.at[pair_v.at[pl.ds(i * CHUNK, CHUNK)]],
                rows[i % 2],
                gsem[i % 2],
            )

        def start_out(i):
            return pltpu.async_copy(
                comp[i % 2], out_hbm.at[pl.ds(base + i * CHUNK, CHUNK)], osem[i % 2]
            )

        def compact(i):
            src = rows[i % 2]
            dst = comp[i % 2]

            def group(gi, carry):
                r0 = gi * LANES
                idx16 = idx_v[pl.ds(i * CHUNK + r0, LANES)]
                for j in range(LANES):
                    off = (idx16[j] & 1) * EMBED_DIM
                    for k in range(EMBED_DIM // LANES):
                        dst[r0 + j, pl.ds(k * LANES, LANES)] = src[
                            r0 + j, pl.ds(off + k * LANES, LANES)
                        ]
                return carry

            lax.fori_loop(0, CHUNK // LANES, group, 0)

        g = [None] * N_CHUNKS
        o = [None] * N_CHUNKS
        g[0] = start_gather(0)
        g[1] = start_gather(1)
        for i in range(N_CHUNKS):
            g[i].wait()
            if i >= 2:
                o[i - 2].wait()
            compact(i)
            o[i] = start_out(i)
            if i + 2 < N_CHUNKS:
                g[i + 2] = start_gather(i + 2)
        o[N_CHUNKS - 2].wait()
        o[N_CHUNKS - 1].wait()

    return gather


def kernel(IX, weight):
    b, t = IX.shape
    total = b * t
    table = _relayout_table(weight.T)
    idx = IX.reshape(-1).astype(jnp.int32)
    out = _make_gather(total)(table, idx)
    return out.reshape(b, t, EMBED_DIM)
